# baseline (speedup 1.0000x reference)
import functools

import jax
import jax.numpy as jnp
from jax import lax
from jax.experimental import pallas as pl
from jax.experimental.pallas import tpu as pltpu
from jax.experimental.pallas import tpu_sc as plsc

_DIM = 64
_NC = 2
_NS = 16
_NW = _NC * _NS

_VOCAB = 1_000_000
_FULL_BLOCKS = _VOCAB // 128
_PACK_ROWS = _VOCAB // 2
_TC_PARAMS = pltpu.CompilerParams(
    use_tc_tiling_on_sc=True, needs_layout_passes=False)


def _iota16():
    return lax.iota(jnp.int32, 16)


@functools.lru_cache(maxsize=None)
def _build_pack():
    mesh = plsc.VectorSubcoreMesh(core_axis_name="c", subcore_axis_name="s")
    ngroups = (_FULL_BLOCKS + _NW - 1) // _NW

    @functools.partial(
        pl.kernel,
        mesh=mesh,
        out_type=jax.ShapeDtypeStruct((_PACK_ROWS, 128), jnp.float32),
        scratch_types=[
            [pltpu.VMEM((_DIM, 129), jnp.float32) for _ in range(2)],
            [pltpu.VMEM((_DIM, 128), jnp.float32) for _ in range(2)],
            [pltpu.SemaphoreType.DMA for _ in range(2)],
            [pltpu.SemaphoreType.DMA for _ in range(2)],
        ],
        compiler_params=_TC_PARAMS,
    )
    def pack(tt_hbm, out_hbm, blk, ob, sem_r, sem_w):
        wid = lax.axis_index("s") * _NC + lax.axis_index("c")
        row16 = [(16 * cg + _iota16()) & 63 for cg in range(8)]
        cvec16 = [16 * cg + _iota16() for cg in range(8)]
        zeros16 = jnp.zeros((16,), jnp.int32)

        def load(j, b):
            pltpu.async_copy(
                tt_hbm.at[:, pl.ds(j * 128, 128)],
                blk[b].at[:, pl.ds(0, 128)], sem_r[b])

        def transpose_block(b):
            @plsc.parallel_loop(0, _DIM, unroll=8)
            def _(p):
                col0 = zeros16 + 2 * p
                col1 = col0 + 1
                psplat = zeros16 + p
                for cg in range(8):
                    val = plsc.load_gather(
                        blk[b], [row16[cg], col0 if cg < 4 else col1])
                    plsc.store_scatter(ob[b], [psplat, cvec16[cg]], val)

        def store(j, b):
            pltpu.async_copy(ob[b], out_hbm.at[pl.ds(j * 64, 64), :], sem_w[b])

        def wait_r(j, b):
            pltpu.make_async_copy(
                tt_hbm.at[:, pl.ds(j * 128, 128)],
                blk[b].at[:, pl.ds(0, 128)], sem_r[b]).wait()

        def wait_w(j, b):
            pltpu.make_async_copy(
                ob[b], out_hbm.at[pl.ds(j * 64, 64), :], sem_w[b]).wait()

        j0 = wid
        j1 = wid + _NW

        @pl.when(j0 < _FULL_BLOCKS)
        def _():
            load(j0, 0)

        @pl.when(j1 < _FULL_BLOCKS)
        def _():
            load(j1, 1)

        def body(g, carry):
            for sub in range(2):
                j = (2 * g + sub) * _NW + wid
                jn = j + 2 * _NW

                @pl.when(j < _FULL_BLOCKS)
                def _():
                    wait_r(j, sub)

                    @pl.when(g > 0)
                    def _():
                        wait_w(j - 2 * _NW, sub)

                    transpose_block(sub)
                    store(j, sub)

                    @pl.when(jn < _FULL_BLOCKS)
                    def _():
                        load(jn, sub)

            return carry

        lax.fori_loop(0, (ngroups + 1) // 2, body, 0)

        nblk = (_FULL_BLOCKS - wid + _NW - 1) // _NW
        for p in range(2):
            i_p = jnp.where((nblk - 1) % 2 == p, nblk - 1, nblk - 2)

            @pl.when(i_p >= 0)
            def _():
                wait_w(i_p * _NW + wid, p)

    return pack


@functools.lru_cache(maxsize=None)
def _build_gather(batch: int, hist: int):
    assert batch % _NW == 0
    bpw = batch // _NW
    mesh = plsc.VectorSubcoreMesh(core_axis_name="c", subcore_axis_name="s")

    @functools.partial(
        pl.kernel,
        mesh=mesh,
        out_type=jax.ShapeDtypeStruct((hist, _DIM, batch), jnp.float32),
        scratch_types=[
            [pltpu.VMEM((bpw,), jnp.int32) for _ in range(2)],
            [pltpu.VMEM((bpw,), jnp.int32) for _ in range(2)],
            [pltpu.VMEM((bpw,), jnp.int32) for _ in range(2)],
            [pltpu.VMEM((bpw, 128), jnp.float32) for _ in range(2)],
            [pltpu.VMEM((_DIM, 129), jnp.float32) for _ in range(2)],
            [pltpu.SemaphoreType.DMA for _ in range(2)],
            [pltpu.SemaphoreType.DMA for _ in range(2)],
            [pltpu.SemaphoreType.DMA for _ in range(2)],
        ],
        compiler_params=_TC_PARAMS,
    )
    def gat(idx_hbm, packed_hbm, out_hbm, idxg, voff, idxr, g, ob, sem_i, sem_g, sem_w):
        wid = lax.axis_index("s") * _NC + lax.axis_index("c")
        b0 = wid * bpw
        iota = _iota16()
        zeros16 = jnp.zeros((16,), jnp.int32)

        def stage_a(h, b):
            pltpu.sync_copy(idx_hbm.at[pl.ds(h * batch + b0, bpw)], idxr[b])
            for kg in range(8):
                v = idxr[b][pl.ds(16 * kg, 16)]
                idxg[b][pl.ds(16 * kg, 16)] = v >> 1
                voff[b][pl.ds(16 * kg, 16)] = (v & 1) << 6
            pltpu.async_copy(packed_hbm.at[idxg[b]], g[b], sem_g[b])

        def fill(b):
            @plsc.parallel_loop(0, bpw, unroll=4)
            def _(k):
                ksplat = zeros16 + k
                voffk = plsc.load_gather(voff[b], [ksplat])
                for dg in range(4):
                    dvec = iota + 16 * dg
                    val = plsc.load_gather(g[b], [ksplat, voffk + dvec])
                    plsc.store_scatter(ob[b], [dvec, ksplat], val)

        def store(h, b):
            pltpu.async_copy(
                ob[b].at[:, pl.ds(0, bpw)],
                out_hbm.at[h, :, pl.ds(b0, bpw)], sem_w[b])

        def wait_g(b):
            pltpu.make_async_copy(packed_hbm.at[idxg[b]], g[b], sem_g[b]).wait()

        def wait_w(h, b):
            pltpu.make_async_copy(
                ob[b].at[:, pl.ds(0, bpw)],
                out_hbm.at[h, :, pl.ds(b0, bpw)], sem_w[b]).wait()

        stage_a(0, 0)
        stage_a(1, 1)

        def body(gg, carry):
            for sub in range(2):
                h = 2 * gg + sub
                wait_g(sub)

                @pl.when(gg > 0)
                def _():
                    wait_w(h - 2, sub)

                fill(sub)
                store(h, sub)

                @pl.when(h + 2 < hist)
                def _():
                    stage_a(h + 2, sub)

            return carry

        lax.fori_loop(0, hist // 2, body, 0)
        wait_w(hist - 2, 0)
        wait_w(hist - 1, 1)

    return gat


def kernel(x, table):
    batch, hist = x.shape
    vocab = table.shape[0]
    assert vocab == _VOCAB

    tt = table.T
    packed = _build_pack()(tt)
    tail = table[_FULL_BLOCKS * 128:, :].reshape(32, 128)
    packed = lax.dynamic_update_slice(packed, tail, (_FULL_BLOCKS * 64, 0))

    idx_hb = x.T.reshape(hist * batch).astype(jnp.int32)
    out_hdb = _build_gather(batch, hist)(idx_hb, packed)
    return out_hdb.transpose(2, 0, 1)


# speedup vs baseline: 1.3316x; 1.3316x over previous
import functools

import jax
import jax.numpy as jnp
from jax import lax
from jax.experimental import pallas as pl
from jax.experimental.pallas import tpu as pltpu
from jax.experimental.pallas import tpu_sc as plsc

_DIM = 64
_NC = 2
_NS = 16
_NW = _NC * _NS
_CHUNK = 640
_NBUF = 2


@functools.lru_cache(maxsize=None)
def _build(total_rows: int, vocab: int):
    assert total_rows % (_NW * _CHUNK) == 0
    b_per_w = total_rows // _NW
    nchunks = b_per_w // _CHUNK
    assert nchunks % _NBUF == 0 and nchunks // _NBUF >= 2
    ngroups = nchunks // _NBUF - 1
    mesh = plsc.VectorSubcoreMesh(core_axis_name="c", subcore_axis_name="s")

    @functools.partial(
        pl.kernel,
        mesh=mesh,
        out_type=jax.ShapeDtypeStruct((total_rows, _DIM), jnp.float32),
        scratch_types=[
            pltpu.VMEM((b_per_w,), jnp.int32),
            [pltpu.VMEM((_CHUNK, _DIM), jnp.float32) for _ in range(_NBUF)],
            [pltpu.SemaphoreType.DMA for _ in range(_NBUF)],
            [pltpu.SemaphoreType.DMA for _ in range(_NBUF)],
        ],
        compiler_params=pltpu.CompilerParams(use_tc_tiling_on_sc=False),
    )
    def emb(x_hbm, table_hbm, out_hbm, idx_all, rows, sem_g, sem_w):
        wid = lax.axis_index("s") * _NC + lax.axis_index("c")
        base = wid * b_per_w

        pltpu.sync_copy(x_hbm.at[pl.ds(base, b_per_w)], idx_all)

        def idx_of(i):
            return idx_all.at[pl.ds(i * _CHUNK, _CHUNK)]

        for b in range(_NBUF):
            pltpu.async_copy(table_hbm.at[idx_of(b)], rows[b], sem_g[b])

        def body(g, carry):
            for b in range(_NBUF):
                i_w = g * _NBUF + b
                i_n = i_w + _NBUF
                pltpu.make_async_copy(table_hbm.at[idx_of(i_w)], rows[b], sem_g[b]).wait()
                w = pltpu.async_copy(
                    rows[b], out_hbm.at[pl.ds(base + i_w * _CHUNK, _CHUNK)], sem_w[b])
                w.wait()
                pltpu.async_copy(table_hbm.at[idx_of(i_n)], rows[b], sem_g[b])
            return carry

        lax.fori_loop(0, ngroups, body, 0)

        last = ngroups * _NBUF
        for b in range(_NBUF):
            pltpu.make_async_copy(table_hbm.at[idx_of(last + b)], rows[b], sem_g[b]).wait()
            pltpu.async_copy(
                rows[b], out_hbm.at[pl.ds(base + (last + b) * _CHUNK, _CHUNK)], sem_w[b])
        for b in range(_NBUF):
            pltpu.make_async_copy(
                rows[b], out_hbm.at[pl.ds(base + (last + b) * _CHUNK, _CHUNK)], sem_w[b]
            ).wait()

    return emb


def kernel(x, table):
    batch, hist = x.shape
    total = batch * hist
    flat = x.reshape(total).astype(jnp.int32)
    out = _build(total, table.shape[0])(flat, table)
    return out.reshape(batch, hist, _DIM)
